# baseline (speedup 1.0000x reference)
import functools

import jax
import jax.numpy as jnp
from jax import lax
from jax.experimental import pallas as pl
from jax.experimental.pallas import tpu as pltpu
from jax.experimental.pallas import tpu_sc as plsc

N = 10000
F_IN = 128
H0 = 8
C0 = 16
D0 = H0 * C0
C1 = 16
TW0 = D0 + 8
TW1 = C1 + 8

NC = 2
NS = 16
L = 16
NW = NC * NS
EB = 128

NP = 10112
TCB = 128
_GRID = NP // TCB

f32 = jnp.float32
i32 = jnp.int32


def _tc_a_body(x_ref, w0_ref, as_ref, ad_ref, t0_ref, d0_ref):
    h = jnp.dot(x_ref[...], w0_ref[...], preferred_element_type=f32)
    a_src = jnp.dot(h, as_ref[...], preferred_element_type=f32)
    t0_ref[...] = jnp.concatenate([h, a_src], axis=1)
    d0_ref[...] = jnp.dot(h, ad_ref[...], preferred_element_type=f32)


def _tc_a(xp, w0, as0, ad0):
    return pl.pallas_call(
        _tc_a_body,
        grid=(_GRID,),
        in_specs=[
            pl.BlockSpec((TCB, F_IN), lambda i: (i, 0)),
            pl.BlockSpec((F_IN, D0), lambda i: (0, 0)),
            pl.BlockSpec((D0, 8), lambda i: (0, 0)),
            pl.BlockSpec((D0, L), lambda i: (0, 0)),
        ],
        out_specs=[
            pl.BlockSpec((TCB, TW0), lambda i: (i, 0)),
            pl.BlockSpec((TCB, L), lambda i: (i, 0)),
        ],
        out_shape=[
            jax.ShapeDtypeStruct((NP, TW0), f32),
            jax.ShapeDtypeStruct((NP, L), f32),
        ],
    )(xp, w0, as0, ad0)


def _make_sc_edge(width, heads, n_batches):
    tw = width + 8
    rows_per = NP // NS
    nb2 = n_batches // 2

    mesh = plsc.VectorSubcoreMesh(core_axis_name="c", subcore_axis_name="s")

    @functools.partial(
        pl.kernel,
        out_type=jax.ShapeDtypeStruct((NC * NP, tw), f32),
        mesh=mesh,
        compiler_params=pltpu.CompilerParams(
            needs_layout_passes=False, use_tc_tiling_on_sc=False),
        scratch_types=[
            pltpu.VMEM_SHARED((NP, tw), f32),
            pltpu.VMEM((2, 2, EB), i32),
            pltpu.VMEM((2, EB, tw), f32),
            pltpu.VMEM((2, EB, L), f32),
            pltpu.SemaphoreType.DMA,
            pltpu.SemaphoreType.DMA,
        ],
    )
    def sc_edge(tbl, dtbl, eidx, zeros_hbm, out,
                acc, eidx_v, rows, ad, sem0, sem1):
        cid = lax.axis_index("c")
        sid = lax.axis_index("s")
        wid = cid * NS + sid
        r0 = sid * rows_per
        sems = (sem0, sem1)

        pltpu.sync_copy(zeros_hbm.at[pl.ds(r0, rows_per)],
                        acc.at[pl.ds(r0, rows_per)])
        plsc.subcore_barrier()

        bbase = wid * n_batches

        def fire(slot, bi):
            pltpu.sync_copy(eidx.at[bi], eidx_v.at[slot])
            pltpu.async_copy(tbl.at[eidx_v.at[slot, 0]], rows.at[slot],
                             sems[slot])
            pltpu.async_copy(dtbl.at[eidx_v.at[slot, 1]], ad.at[slot],
                             sems[slot])

        def drain(slot):
            pltpu.make_async_copy(tbl.at[eidx_v.at[slot, 0]], rows.at[slot],
                                  sems[slot]).wait()
            pltpu.make_async_copy(dtbl.at[eidx_v.at[slot, 1]], ad.at[slot],
                                  sems[slot]).wait()

        def compute(slot):
            r2 = rows.at[slot]
            a2 = ad.at[slot]
            @plsc.parallel_loop(0, EB, step=L, unroll=2)
            def grp(e0):
                eids = jnp.full((L,), 0, i32) + e0 + lax.iota(i32, L)
                ps = []
                for h in range(heads):
                    a_s = plsc.load_gather(r2, [eids, jnp.full((L,), width + h, i32)])
                    a_d = plsc.load_gather(a2, [eids, jnp.full((L,), h, i32)])
                    s = a_s + a_d
                    ps.append(jnp.exp(jnp.maximum(s, 0.2 * s)))
                for h in range(heads):
                    plsc.store_scatter(r2, [eids, jnp.full((L,), width + h, i32)], ps[h])

            @plsc.parallel_loop(0, EB, unroll=8)
            def scale(e):
                evec = jnp.full((L,), 0, i32) + e
                ms = [plsc.load_gather(r2, [evec, jnp.full((L,), width + h, i32)])
                      for h in range(heads)]
                for h in range(heads):
                    chunk = rows[slot, e, pl.ds(h * L, L)]
                    rows[slot, e, pl.ds(h * L, L)] = chunk * ms[h]

        def scatter(slot):
            pltpu.sync_copy(rows.at[slot], acc.at[eidx_v.at[slot, 1]], add=True)

        fire(0, bbase)

        def outer(t, carry):
            b0 = bbase + 2 * t
            fire(1, b0 + 1)
            drain(0)
            compute(0)
            scatter(0)

            @pl.when(t + 1 < nb2)
            def _():
                fire(0, b0 + 2)

            drain(1)
            compute(1)
            scatter(1)
            return carry

        lax.fori_loop(0, nb2, outer, 0)
        plsc.subcore_barrier()

        obase = cid * NP + r0
        pltpu.sync_copy(acc.at[pl.ds(r0, rows_per)],
                        out.at[pl.ds(obase, rows_per)])

    return sc_edge


def _tc_b_body(p0_ref, p1_ref, m0_ref, w1_ref, as1_ref, ad1_ref, b0_ref,
               t1_ref, d1_ref):
    s = p0_ref[...] + p1_ref[...]
    num = s[:, :D0]
    den_w = jnp.dot(s, m0_ref[...], preferred_element_type=f32)
    z = num / (den_w + 1e-16) + b0_ref[...]
    g = jnp.where(z > 0, z, jnp.exp(z) - 1.0)
    h1 = jnp.dot(g, w1_ref[...], preferred_element_type=f32)
    a_src = jnp.dot(h1, as1_ref[...], preferred_element_type=f32)
    t1_ref[...] = jnp.concatenate([h1, a_src], axis=1)
    d1_ref[...] = jnp.dot(h1, ad1_ref[...], preferred_element_type=f32)


def _tc_b(p0, p1, m0, w1, as1, ad1, b0row):
    return pl.pallas_call(
        _tc_b_body,
        grid=(_GRID,),
        in_specs=[
            pl.BlockSpec((TCB, TW0), lambda i: (i, 0)),
            pl.BlockSpec((TCB, TW0), lambda i: (i, 0)),
            pl.BlockSpec((TW0, D0), lambda i: (0, 0)),
            pl.BlockSpec((D0, C1), lambda i: (0, 0)),
            pl.BlockSpec((C1, 8), lambda i: (0, 0)),
            pl.BlockSpec((C1, L), lambda i: (0, 0)),
            pl.BlockSpec((1, D0), lambda i: (0, 0)),
        ],
        out_specs=[
            pl.BlockSpec((TCB, TW1), lambda i: (i, 0)),
            pl.BlockSpec((TCB, L), lambda i: (i, 0)),
        ],
        out_shape=[
            jax.ShapeDtypeStruct((NP, TW1), f32),
            jax.ShapeDtypeStruct((NP, L), f32),
        ],
    )(p0, p1, m0, w1, as1, ad1, b0row)


def _tc_c_body(p0_ref, p1_ref, m1_ref, b1_ref, out_ref):
    s = p0_ref[...] + p1_ref[...]
    den = jnp.dot(s, m1_ref[...], preferred_element_type=f32)
    out_ref[...] = s[:, :C1] / (den + 1e-16) + b1_ref[...]


def _tc_c(p0, p1, m1, b1row):
    return pl.pallas_call(
        _tc_c_body,
        grid=(_GRID,),
        in_specs=[
            pl.BlockSpec((TCB, TW1), lambda i: (i, 0)),
            pl.BlockSpec((TCB, TW1), lambda i: (i, 0)),
            pl.BlockSpec((TW1, C1), lambda i: (0, 0)),
            pl.BlockSpec((1, C1), lambda i: (0, 0)),
        ],
        out_specs=pl.BlockSpec((TCB, C1), lambda i: (i, 0)),
        out_shape=jax.ShapeDtypeStruct((NP, C1), f32),
    )(p0, p1, m1, b1row)


@jax.jit
def kernel(x, edge_index, W0, att_src0, att_dst0, b0, W1, att_src1, att_dst1, b1):
    E = edge_index.shape[1]
    e2 = E + N
    n_batches = -(-e2 // (NW * EB))
    n_batches += n_batches % 2
    ep = n_batches * NW * EB

    xp = jnp.zeros((NP, F_IN), f32).at[:N].set(x)
    loop = jnp.arange(N, dtype=i32)
    fill = jnp.full((ep - e2,), N, dtype=i32)
    src = jnp.concatenate([edge_index[0].astype(i32), loop, fill])
    dst = jnp.concatenate([edge_index[1].astype(i32), loop, fill])
    eidx = jnp.stack([src.reshape(-1, EB), dst.reshape(-1, EB)], axis=1)

    rows128 = jnp.arange(D0)
    head_of = rows128[:, None] // C0
    sel8 = (head_of == jnp.arange(8)[None, :]).astype(f32)
    sel16 = (head_of == jnp.arange(L)[None, :]).astype(f32)
    as0 = att_src0.reshape(-1)[:, None] * sel8
    ad0 = att_dst0.reshape(-1)[:, None] * sel16
    as1 = att_src1.reshape(-1)[:, None] * (jnp.arange(8)[None, :] == 0)
    ad1 = att_dst1.reshape(-1)[:, None] * (jnp.arange(L)[None, :] == 0)
    m0 = ((jnp.arange(TW0)[:, None] - D0) == jnp.arange(D0)[None, :] // C0)
    m0 = m0.astype(f32)
    m1 = (jnp.arange(TW1)[:, None] == C1).astype(f32) * jnp.ones((1, C1), f32)
    b0row = b0.reshape(1, D0).astype(f32)
    b1row = b1.reshape(1, C1).astype(f32)

    z0 = jnp.zeros((NP, TW0), f32)
    z1 = jnp.zeros((NP, TW1), f32)

    t0, d0t = _tc_a(xp, W0.astype(f32), as0, ad0)
    sc0 = _make_sc_edge(D0, H0, n_batches)
    part0 = sc0(t0, d0t, eidx, z0)
    t1, d1t = _tc_b(part0[:NP], part0[NP:], m0, W1.astype(f32), as1.astype(f32),
                    ad1.astype(f32), b0row)

    sc1 = _make_sc_edge(C1, 1, n_batches)
    part1 = sc1(t1, d1t, eidx, z1)
    outp = _tc_c(part1[:NP], part1[NP:], m1, b1row)
    return outp[:N]


# speedup vs baseline: 1.0903x; 1.0903x over previous
import functools

import jax
import jax.numpy as jnp
from jax import lax
from jax.experimental import pallas as pl
from jax.experimental.pallas import tpu as pltpu
from jax.experimental.pallas import tpu_sc as plsc

N = 10000
F_IN = 128
H0 = 8
C0 = 16
D0 = H0 * C0
C1 = 16
TW0 = D0 + 8
TW1 = C1 + 8

NC = 2
NS = 16
L = 16
NW = NC * NS
EB = 128

NP = 10112
TCB = 632
_GRID = NP // TCB

f32 = jnp.float32
i32 = jnp.int32


def _tc_a_body(x_ref, w0_ref, as_ref, ad_ref, t0_ref, d0_ref):
    h = jnp.dot(x_ref[...], w0_ref[...], preferred_element_type=f32)
    a_src = jnp.dot(h, as_ref[...], preferred_element_type=f32)
    t0_ref[...] = jnp.concatenate([h, a_src], axis=1)
    d0_ref[...] = jnp.dot(h, ad_ref[...], preferred_element_type=f32)


def _tc_a(xp, w0, as0, ad0):
    return pl.pallas_call(
        _tc_a_body,
        grid=(_GRID,),
        in_specs=[
            pl.BlockSpec((TCB, F_IN), lambda i: (i, 0)),
            pl.BlockSpec((F_IN, D0), lambda i: (0, 0)),
            pl.BlockSpec((D0, 8), lambda i: (0, 0)),
            pl.BlockSpec((D0, L), lambda i: (0, 0)),
        ],
        out_specs=[
            pl.BlockSpec((TCB, TW0), lambda i: (i, 0)),
            pl.BlockSpec((TCB, L), lambda i: (i, 0)),
        ],
        out_shape=[
            jax.ShapeDtypeStruct((NP, TW0), f32),
            jax.ShapeDtypeStruct((NP, L), f32),
        ],
    )(xp, w0, as0, ad0)


def _make_sc_edge(width, heads, n_batches):
    tw = width + 8
    rows_per = NP // NS
    nb2 = n_batches // 2

    mesh = plsc.VectorSubcoreMesh(core_axis_name="c", subcore_axis_name="s")

    @functools.partial(
        pl.kernel,
        out_type=jax.ShapeDtypeStruct((NC * NP, tw), f32),
        mesh=mesh,
        compiler_params=pltpu.CompilerParams(
            needs_layout_passes=False, use_tc_tiling_on_sc=False),
        scratch_types=[
            pltpu.VMEM_SHARED((NP, tw), f32),
            pltpu.VMEM((2, 2, EB), i32),
            pltpu.VMEM((2, EB, tw), f32),
            pltpu.VMEM((2, EB, L), f32),
            pltpu.SemaphoreType.DMA,
            pltpu.SemaphoreType.DMA,
        ],
    )
    def sc_edge(tbl, dtbl, eidx, zeros_hbm, out,
                acc, eidx_v, rows, ad, sem0, sem1):
        cid = lax.axis_index("c")
        sid = lax.axis_index("s")
        wid = cid * NS + sid
        r0 = sid * rows_per
        sems = (sem0, sem1)

        pltpu.sync_copy(zeros_hbm.at[pl.ds(r0, rows_per)],
                        acc.at[pl.ds(r0, rows_per)])
        plsc.subcore_barrier()

        def fire(slot, bi):
            pltpu.sync_copy(eidx.at[bi], eidx_v.at[slot])
            pltpu.async_copy(tbl.at[eidx_v.at[slot, 0]], rows.at[slot],
                             sems[slot])
            pltpu.async_copy(dtbl.at[eidx_v.at[slot, 1]], ad.at[slot],
                             sems[slot])

        def drain(slot):
            pltpu.make_async_copy(tbl.at[eidx_v.at[slot, 0]], rows.at[slot],
                                  sems[slot]).wait()
            pltpu.make_async_copy(dtbl.at[eidx_v.at[slot, 1]], ad.at[slot],
                                  sems[slot]).wait()

        def compute(slot):
            r2 = rows.at[slot]
            a2 = ad.at[slot]
            for g in range(EB // L):
                eids = jnp.full((L,), g * L, i32) + lax.iota(i32, L)
                ps = []
                for h in range(heads):
                    a_s = plsc.load_gather(r2, [eids, jnp.full((L,), width + h, i32)])
                    a_d = plsc.load_gather(a2, [eids, jnp.full((L,), h, i32)])
                    s = a_s + a_d
                    ps.append(jnp.exp(jnp.maximum(s, 0.2 * s)))
                for h in range(heads):
                    plsc.store_scatter(r2, [eids, jnp.full((L,), width + h, i32)], ps[h])

            @plsc.parallel_loop(0, EB, unroll=4)
            def scale(e):
                evec = jnp.full((L,), 0, i32) + e
                ms = [plsc.load_gather(r2, [evec, jnp.full((L,), width + h, i32)])
                      for h in range(heads)]
                for h in range(heads):
                    chunk = rows[slot, e, pl.ds(h * L, L)]
                    rows[slot, e, pl.ds(h * L, L)] = chunk * ms[h]

        def scatter(slot):
            pltpu.sync_copy(rows.at[slot], acc.at[eidx_v.at[slot, 1]], add=True)

        fire(0, wid)

        def outer(t, carry):
            k0 = 2 * t
            fire(1, wid + (k0 + 1) * NW)
            drain(0)
            compute(0)
            scatter(0)

            @pl.when(t + 1 < nb2)
            def _():
                fire(0, wid + (k0 + 2) * NW)

            drain(1)
            compute(1)
            scatter(1)
            return carry

        lax.fori_loop(0, nb2, outer, 0)
        plsc.subcore_barrier()

        obase = cid * NP + r0
        pltpu.sync_copy(acc.at[pl.ds(r0, rows_per)],
                        out.at[pl.ds(obase, rows_per)])

    return sc_edge


def _tc_b_body(p0_ref, p1_ref, m0_ref, w1_ref, as1_ref, ad1_ref, b0_ref,
               t1_ref, d1_ref):
    s = p0_ref[...] + p1_ref[...]
    num = s[:, :D0]
    den_w = jnp.dot(s, m0_ref[...], preferred_element_type=f32)
    z = num / (den_w + 1e-16) + b0_ref[...]
    g = jnp.where(z > 0, z, jnp.exp(z) - 1.0)
    h1 = jnp.dot(g, w1_ref[...], preferred_element_type=f32)
    a_src = jnp.dot(h1, as1_ref[...], preferred_element_type=f32)
    t1_ref[...] = jnp.concatenate([h1, a_src], axis=1)
    d1_ref[...] = jnp.dot(h1, ad1_ref[...], preferred_element_type=f32)


def _tc_b(p0, p1, m0, w1, as1, ad1, b0row):
    return pl.pallas_call(
        _tc_b_body,
        grid=(_GRID,),
        in_specs=[
            pl.BlockSpec((TCB, TW0), lambda i: (i, 0)),
            pl.BlockSpec((TCB, TW0), lambda i: (i + _GRID, 0)),
            pl.BlockSpec((TW0, D0), lambda i: (0, 0)),
            pl.BlockSpec((D0, C1), lambda i: (0, 0)),
            pl.BlockSpec((C1, 8), lambda i: (0, 0)),
            pl.BlockSpec((C1, L), lambda i: (0, 0)),
            pl.BlockSpec((1, D0), lambda i: (0, 0)),
        ],
        out_specs=[
            pl.BlockSpec((TCB, TW1), lambda i: (i, 0)),
            pl.BlockSpec((TCB, L), lambda i: (i, 0)),
        ],
        out_shape=[
            jax.ShapeDtypeStruct((NP, TW1), f32),
            jax.ShapeDtypeStruct((NP, L), f32),
        ],
    )(p0, p1, m0, w1, as1, ad1, b0row)


def _tc_c_body(p0_ref, p1_ref, m1_ref, b1_ref, out_ref):
    s = p0_ref[...] + p1_ref[...]
    den = jnp.dot(s, m1_ref[...], preferred_element_type=f32)
    out_ref[...] = s[:, :C1] / (den + 1e-16) + b1_ref[...]


def _tc_c(p0, p1, m1, b1row):
    return pl.pallas_call(
        _tc_c_body,
        grid=(_GRID,),
        in_specs=[
            pl.BlockSpec((TCB, TW1), lambda i: (i, 0)),
            pl.BlockSpec((TCB, TW1), lambda i: (i + _GRID, 0)),
            pl.BlockSpec((TW1, C1), lambda i: (0, 0)),
            pl.BlockSpec((1, C1), lambda i: (0, 0)),
        ],
        out_specs=pl.BlockSpec((TCB, C1), lambda i: (i, 0)),
        out_shape=jax.ShapeDtypeStruct((NP, C1), f32),
    )(p0, p1, m1, b1row)


@jax.jit
def kernel(x, edge_index, W0, att_src0, att_dst0, b0, W1, att_src1, att_dst1, b1):
    E = edge_index.shape[1]
    e2 = E + N
    n_batches = -(-e2 // (NW * EB))
    n_batches += n_batches % 2
    ep = n_batches * NW * EB

    xp = jnp.zeros((NP, F_IN), f32).at[:N].set(x)
    loop = jnp.arange(N, dtype=i32)
    fill_src = jnp.full((ep - e2,), N, dtype=i32)
    fill_dst = N + 1 + (jnp.arange(ep - e2, dtype=i32) % (NP - N - 1))
    src = jnp.concatenate([edge_index[0].astype(i32), loop, fill_src])
    dst = jnp.concatenate([edge_index[1].astype(i32), loop, fill_dst])
    eidx = jnp.stack([src.reshape(-1, EB), dst.reshape(-1, EB)], axis=1)

    rows128 = jnp.arange(D0)
    head_of = rows128[:, None] // C0
    sel8 = (head_of == jnp.arange(8)[None, :]).astype(f32)
    sel16 = (head_of == jnp.arange(L)[None, :]).astype(f32)
    as0 = att_src0.reshape(-1)[:, None] * sel8
    ad0 = att_dst0.reshape(-1)[:, None] * sel16
    as1 = att_src1.reshape(-1)[:, None] * (jnp.arange(8)[None, :] == 0)
    ad1 = att_dst1.reshape(-1)[:, None] * (jnp.arange(L)[None, :] == 0)
    m0 = ((jnp.arange(TW0)[:, None] - D0) == jnp.arange(D0)[None, :] // C0)
    m0 = m0.astype(f32)
    m1 = (jnp.arange(TW1)[:, None] == C1).astype(f32) * jnp.ones((1, C1), f32)
    b0row = b0.reshape(1, D0).astype(f32)
    b1row = b1.reshape(1, C1).astype(f32)

    z0 = jnp.zeros((NP, TW0), f32)
    z1 = jnp.zeros((NP, TW1), f32)

    t0, d0t = _tc_a(xp, W0.astype(f32), as0, ad0)
    sc0 = _make_sc_edge(D0, H0, n_batches)
    part0 = sc0(t0, d0t, eidx, z0)
    t1, d1t = _tc_b(part0, part0, m0, W1.astype(f32), as1.astype(f32),
                    ad1.astype(f32), b0row)

    sc1 = _make_sc_edge(C1, 1, n_batches)
    part1 = sc1(t1, d1t, eidx, z1)
    outp = _tc_c(part1, part1, m1, b1row)
    return outp[:N]


# speedup vs baseline: 1.2276x; 1.1260x over previous
import functools

import jax
import jax.numpy as jnp
from jax import lax
from jax.experimental import pallas as pl
from jax.experimental.pallas import tpu as pltpu
from jax.experimental.pallas import tpu_sc as plsc

N = 10000
F_IN = 128
H0 = 8
C0 = 16
D0 = H0 * C0
C1 = 16
TW0 = D0 + 8
TW1 = C1 + 8

NC = 2
NS = 16
L = 16
NW = NC * NS
EB = 128

NP = 10112
TCB = 632
_GRID = NP // TCB

f32 = jnp.float32
i32 = jnp.int32


def _tc_a_body(x_ref, w0_ref, as_ref, ad_ref, t0_ref, d0_ref):
    h = jnp.dot(x_ref[...], w0_ref[...], preferred_element_type=f32)
    a_src = jnp.dot(h, as_ref[...], preferred_element_type=f32)
    t0_ref[...] = jnp.concatenate([h, a_src], axis=1)
    d0_ref[...] = jnp.dot(h, ad_ref[...], preferred_element_type=f32)


def _tc_a(xp, w0, as0, ad0):
    return pl.pallas_call(
        _tc_a_body,
        grid=(_GRID,),
        in_specs=[
            pl.BlockSpec((TCB, F_IN), lambda i: (i, 0)),
            pl.BlockSpec((F_IN, D0), lambda i: (0, 0)),
            pl.BlockSpec((D0, 8), lambda i: (0, 0)),
            pl.BlockSpec((D0, L), lambda i: (0, 0)),
        ],
        out_specs=[
            pl.BlockSpec((TCB, TW0), lambda i: (i, 0)),
            pl.BlockSpec((TCB, L), lambda i: (i, 0)),
        ],
        out_shape=[
            jax.ShapeDtypeStruct((NP, TW0), f32),
            jax.ShapeDtypeStruct((NP, L), f32),
        ],
    )(xp, w0, as0, ad0)


def _make_sc_edge(width, heads, n_batches):
    tw = width + 8
    rows_per = NP // NS
    nb2 = n_batches // 2

    mesh = plsc.VectorSubcoreMesh(core_axis_name="c", subcore_axis_name="s")

    @functools.partial(
        pl.kernel,
        out_type=jax.ShapeDtypeStruct((NC * NP, tw), f32),
        mesh=mesh,
        compiler_params=pltpu.CompilerParams(
            needs_layout_passes=False, use_tc_tiling_on_sc=False),
        scratch_types=[
            pltpu.VMEM_SHARED((NP, tw), f32),
            pltpu.VMEM((2, 2, EB), i32),
            pltpu.VMEM((2, EB, tw), f32),
            pltpu.VMEM((2, EB, L), f32),
            pltpu.SemaphoreType.DMA,
            pltpu.SemaphoreType.DMA,
        ],
    )
    def sc_edge(tbl, dtbl, eidx, zeros_hbm, out,
                acc, eidx_v, rows, ad, sem0, sem1):
        cid = lax.axis_index("c")
        sid = lax.axis_index("s")
        wid = cid * NS + sid
        r0 = sid * rows_per
        sems = (sem0, sem1)

        pltpu.sync_copy(zeros_hbm.at[pl.ds(r0, rows_per)],
                        acc.at[pl.ds(r0, rows_per)])
        plsc.subcore_barrier()

        def fire(slot, bi):
            pltpu.sync_copy(eidx.at[bi], eidx_v.at[slot])
            pltpu.async_copy(tbl.at[eidx_v.at[slot, 0]], rows.at[slot],
                             sems[slot])
            pltpu.async_copy(dtbl.at[eidx_v.at[slot, 1]], ad.at[slot],
                             sems[slot])

        def drain(slot):
            pltpu.make_async_copy(tbl.at[eidx_v.at[slot, 0]], rows.at[slot],
                                  sems[slot]).wait()
            pltpu.make_async_copy(dtbl.at[eidx_v.at[slot, 1]], ad.at[slot],
                                  sems[slot]).wait()

        def compute(slot):
            r2 = rows.at[slot]
            a2 = ad.at[slot]
            for g in range(EB // L):
                eids = jnp.full((L,), g * L, i32) + lax.iota(i32, L)
                ps = []
                for h in range(heads):
                    a_s = plsc.load_gather(r2, [eids, jnp.full((L,), width + h, i32)])
                    a_d = plsc.load_gather(a2, [eids, jnp.full((L,), h, i32)])
                    s = a_s + a_d
                    ps.append(jnp.exp(jnp.maximum(s, 0.2 * s)))
                for h in range(heads):
                    plsc.store_scatter(r2, [eids, jnp.full((L,), width + h, i32)], ps[h])

            @plsc.parallel_loop(0, EB, unroll=4)
            def scale(e):
                evec = jnp.full((L,), 0, i32) + e
                ms = [plsc.load_gather(r2, [evec, jnp.full((L,), width + h, i32)])
                      for h in range(heads)]
                for h in range(heads):
                    chunk = rows[slot, e, pl.ds(h * L, L)]
                    rows[slot, e, pl.ds(h * L, L)] = chunk * ms[h]

        def scatter(slot):
            pltpu.sync_copy(rows.at[slot], acc.at[eidx_v.at[slot, 1]], add=True)

        bbase = wid * n_batches
        fire(0, bbase)

        def outer(t, carry):
            b0 = bbase + 2 * t
            fire(1, b0 + 1)
            drain(0)
            compute(0)
            scatter(0)

            @pl.when(t + 1 < nb2)
            def _():
                fire(0, b0 + 2)

            drain(1)
            compute(1)
            scatter(1)
            return carry

        lax.fori_loop(0, nb2, outer, 0)
        plsc.subcore_barrier()

        obase = cid * NP + r0
        pltpu.sync_copy(acc.at[pl.ds(r0, rows_per)],
                        out.at[pl.ds(obase, rows_per)])

    return sc_edge


def _tc_b_body(p0_ref, p1_ref, m0_ref, w1_ref, as1_ref, ad1_ref, b0_ref,
               t1_ref, d1_ref):
    s = p0_ref[...] + p1_ref[...]
    num = s[:, :D0]
    den_w = jnp.dot(s, m0_ref[...], preferred_element_type=f32)
    z = num / (den_w + 1e-16) + b0_ref[...]
    g = jnp.where(z > 0, z, jnp.exp(z) - 1.0)
    h1 = jnp.dot(g, w1_ref[...], preferred_element_type=f32)
    a_src = jnp.dot(h1, as1_ref[...], preferred_element_type=f32)
    t1_ref[...] = jnp.concatenate([h1, a_src], axis=1)
    d1_ref[...] = jnp.dot(h1, ad1_ref[...], preferred_element_type=f32)


def _tc_b(p0, p1, m0, w1, as1, ad1, b0row):
    return pl.pallas_call(
        _tc_b_body,
        grid=(_GRID,),
        in_specs=[
            pl.BlockSpec((TCB, TW0), lambda i: (i, 0)),
            pl.BlockSpec((TCB, TW0), lambda i: (i + _GRID, 0)),
            pl.BlockSpec((TW0, D0), lambda i: (0, 0)),
            pl.BlockSpec((D0, C1), lambda i: (0, 0)),
            pl.BlockSpec((C1, 8), lambda i: (0, 0)),
            pl.BlockSpec((C1, L), lambda i: (0, 0)),
            pl.BlockSpec((1, D0), lambda i: (0, 0)),
        ],
        out_specs=[
            pl.BlockSpec((TCB, TW1), lambda i: (i, 0)),
            pl.BlockSpec((TCB, L), lambda i: (i, 0)),
        ],
        out_shape=[
            jax.ShapeDtypeStruct((NP, TW1), f32),
            jax.ShapeDtypeStruct((NP, L), f32),
        ],
    )(p0, p1, m0, w1, as1, ad1, b0row)


def _tc_c_body(p0_ref, p1_ref, m1_ref, b1_ref, out_ref):
    s = p0_ref[...] + p1_ref[...]
    den = jnp.dot(s, m1_ref[...], preferred_element_type=f32)
    out_ref[...] = s[:, :C1] / (den + 1e-16) + b1_ref[...]


def _tc_c(p0, p1, m1, b1row):
    return pl.pallas_call(
        _tc_c_body,
        grid=(_GRID,),
        in_specs=[
            pl.BlockSpec((TCB, TW1), lambda i: (i, 0)),
            pl.BlockSpec((TCB, TW1), lambda i: (i + _GRID, 0)),
            pl.BlockSpec((TW1, C1), lambda i: (0, 0)),
            pl.BlockSpec((1, C1), lambda i: (0, 0)),
        ],
        out_specs=pl.BlockSpec((TCB, C1), lambda i: (i, 0)),
        out_shape=jax.ShapeDtypeStruct((NP, C1), f32),
    )(p0, p1, m1, b1row)


@jax.jit
def kernel(x, edge_index, W0, att_src0, att_dst0, b0, W1, att_src1, att_dst1, b1):
    E = edge_index.shape[1]
    e2 = E + N
    n_batches = -(-e2 // (NW * EB))
    n_batches += n_batches % 2
    ep = n_batches * NW * EB

    xp = jnp.zeros((NP, F_IN), f32).at[:N].set(x)
    loop = jnp.arange(N, dtype=i32)
    fill_src = jnp.full((ep - e2,), N, dtype=i32)
    fill_dst = N + 1 + (jnp.arange(ep - e2, dtype=i32) % (NP - N - 1))
    src = jnp.concatenate([edge_index[0].astype(i32), loop, fill_src])
    dst = jnp.concatenate([edge_index[1].astype(i32), loop, fill_dst])
    eidx = jnp.stack([src.reshape(-1, EB), dst.reshape(-1, EB)], axis=1)

    rows128 = jnp.arange(D0)
    head_of = rows128[:, None] // C0
    sel8 = (head_of == jnp.arange(8)[None, :]).astype(f32)
    sel16 = (head_of == jnp.arange(L)[None, :]).astype(f32)
    as0 = att_src0.reshape(-1)[:, None] * sel8
    ad0 = att_dst0.reshape(-1)[:, None] * sel16
    as1 = att_src1.reshape(-1)[:, None] * (jnp.arange(8)[None, :] == 0)
    ad1 = att_dst1.reshape(-1)[:, None] * (jnp.arange(L)[None, :] == 0)
    m0 = ((jnp.arange(TW0)[:, None] - D0) == jnp.arange(D0)[None, :] // C0)
    m0 = m0.astype(f32)
    m1 = (jnp.arange(TW1)[:, None] == C1).astype(f32) * jnp.ones((1, C1), f32)
    b0row = b0.reshape(1, D0).astype(f32)
    b1row = b1.reshape(1, C1).astype(f32)

    z0 = jnp.zeros((NP, TW0), f32)
    z1 = jnp.zeros((NP, TW1), f32)

    t0, d0t = _tc_a(xp, W0.astype(f32), as0, ad0)
    sc0 = _make_sc_edge(D0, H0, n_batches)
    part0 = sc0(t0, d0t, eidx, z0)
    t1, d1t = _tc_b(part0, part0, m0, W1.astype(f32), as1.astype(f32),
                    ad1.astype(f32), b0row)

    sc1 = _make_sc_edge(C1, 1, n_batches)
    part1 = sc1(t1, d1t, eidx, z1)
    outp = _tc_c(part1, part1, m1, b1row)
    return outp[:N]
